# baseline (speedup 1.0000x reference)
import jax
import jax.numpy as jnp
from jax import lax
from jax.experimental import pallas as pl
from jax.experimental.pallas import tpu as pltpu
from jax.experimental.pallas import tpu_sc as plsc

L = 16
NC = 2
NS = 16
B, C, H, W = 16, 3, 512, 512
NIMG = B * C
IMG_PER_CORE = NIMG // NC
G = 4
NGRP = IMG_PER_CORE // G
ROWS = H // NS
NK = W // L
NB = 256 // L


def _he_body(x_hbm, out_hbm, buf, hist, mhistg, mhist, merge, lut, lutbuf,
             shared_hist, shared_lut, in_sem, out_sem):
    cid = lax.axis_index("c")
    sid = lax.axis_index("s")
    row0 = sid * ROWS
    lane = lax.broadcasted_iota(jnp.int32, (L,), 0)
    lane_base = lane * 256
    ones = jnp.ones((L,), jnp.float32)
    zeros = jnp.zeros((L,), jnp.float32)

    def group(grp, _):
        img0 = cid * IMG_PER_CORE + grp * G

        for g in range(G):
            img = img0 + g
            bi, ci = img // C, img % C
            pv = img - G
            pb, pc = pv // C, pv % C

            @pl.when(grp > 0)
            def _():
                pltpu.make_async_copy(
                    buf.at[g], out_hbm.at[pb, pc, pl.ds(row0, ROWS)],
                    out_sem.at[g]).wait()

            pltpu.async_copy(
                x_hbm.at[bi, ci, pl.ds(row0, ROWS)], buf.at[g],
                in_sem.at[g])

        for g in range(G):
            img = img0 + g
            bi, ci = img // C, img % C
            pltpu.make_async_copy(
                x_hbm.at[bi, ci, pl.ds(row0, ROWS)], buf.at[g],
                in_sem.at[g]).wait()

            @plsc.parallel_loop(0, 256, unroll=8)
            def zero_hist(i):
                hist[pl.ds(i * L, L)] = zeros

            @plsc.parallel_loop(0, ROWS)
            def hist_row(r):
                @plsc.parallel_loop(0, NK, unroll=16)
                def hist_px(k):
                    v = buf[g, r, pl.ds(k * L, L)]
                    xi = (v * 255.0).astype(jnp.int32)
                    plsc.addupdate_scatter(hist, [lane_base + xi], ones)

            @plsc.parallel_loop(0, NB, unroll=2)
            def merge_lanes(c):
                acc = hist[pl.ds(c * L, L)]
                for r in range(1, L):
                    acc = acc + hist[pl.ds(r * 256 + c * L, L)]
                mhistg[g, pl.ds(c * L, L)] = acc

        pltpu.sync_copy(mhistg, shared_hist.at[sid])
        plsc.subcore_barrier()

        @pl.when(sid < G)
        def _():
            g = sid
            pltpu.sync_copy(shared_hist, merge)

            def merge_subcores(c, carry):
                tot, m = carry
                acc = merge[0, g, pl.ds(c * L, L)]
                for r in range(1, NS):
                    acc = acc + merge[r, g, pl.ds(c * L, L)]
                mhist[pl.ds(c * L, L)] = acc
                idx = lane + c * L
                comb = jnp.where(acc != 0.0,
                                 idx * 524288 + acc.astype(jnp.int32),
                                 -1)
                return tot + jnp.sum(acc), jnp.maximum(m, jnp.max(comb))

            tot, m = lax.fori_loop(
                0, NB, merge_subcores,
                (jnp.float32(0.0), jnp.int32(-1)))

            tot_v = jnp.full((L,), tot, jnp.float32)
            last_v = jnp.bitwise_and(jnp.full((L,), m, jnp.int32),
                                     524287).astype(jnp.float32)
            step_v = ((tot_v - last_v) / 255.0).astype(
                jnp.int32).astype(jnp.float32)
            half_v = (step_v * 0.5).astype(jnp.int32).astype(jnp.float32)
            safe_v = jnp.maximum(step_v, 1.0)
            is_id = step_v == 0.0

            def lut_chunk(c, carry_f):
                v = mhist[pl.ds(c * L, L)]
                excl = plsc.cumsum(v) + carry_f - v
                q = ((excl + half_v) / safe_v).astype(
                    jnp.int32).astype(jnp.float32)
                qc = jnp.clip(q, 0.0, 255.0)
                idx_f = (lane + c * L).astype(jnp.float32)
                lut[pl.ds(c * L, L)] = jnp.where(is_id, idx_f, qc) / 255.0
                return carry_f + jnp.sum(v)

            lax.fori_loop(0, NB, lut_chunk, jnp.float32(0.0))
            pltpu.sync_copy(lut, shared_lut.at[pl.ds(g * 256, 256)])

        plsc.subcore_barrier()
        pltpu.sync_copy(shared_lut, lutbuf)

        for g in range(G):
            img = img0 + g
            bi, ci = img // C, img % C

            @plsc.parallel_loop(0, ROWS)
            def gather_row(r):
                @plsc.parallel_loop(0, NK, unroll=16)
                def gather_px(k):
                    v = buf[g, r, pl.ds(k * L, L)]
                    xi = (v * 255.0).astype(jnp.int32) + g * 256
                    buf[g, r, pl.ds(k * L, L)] = plsc.load_gather(
                        lutbuf, [xi])

            pltpu.async_copy(
                buf.at[g], out_hbm.at[bi, ci, pl.ds(row0, ROWS)],
                out_sem.at[g])
        return 0

    lax.fori_loop(0, NGRP, group, 0)

    for g in range(G):
        img = cid * IMG_PER_CORE + (NGRP - 1) * G + g
        bi, ci = img // C, img % C
        pltpu.make_async_copy(
            buf.at[g], out_hbm.at[bi, ci, pl.ds(row0, ROWS)],
            out_sem.at[g]).wait()


_he = pl.kernel(
    _he_body,
    out_type=jax.ShapeDtypeStruct((B, C, H, W), jnp.float32),
    mesh=plsc.VectorSubcoreMesh(core_axis_name="c", subcore_axis_name="s"),
    compiler_params=pltpu.CompilerParams(
        needs_layout_passes=False, use_tc_tiling_on_sc=True),
    scratch_types=[
        pltpu.VMEM((G, ROWS, W), jnp.float32),
        pltpu.VMEM((L * 256,), jnp.float32),
        pltpu.VMEM((G, 256), jnp.float32),
        pltpu.VMEM((256,), jnp.float32),
        pltpu.VMEM((NS, G, 256), jnp.float32),
        pltpu.VMEM((256,), jnp.float32),
        pltpu.VMEM((G * 256,), jnp.float32),
        pltpu.VMEM_SHARED((NS, G, 256), jnp.float32),
        pltpu.VMEM_SHARED((G * 256,), jnp.float32),
        pltpu.SemaphoreType.DMA((G,)),
        pltpu.SemaphoreType.DMA((G,)),
    ],
)


def kernel(x):
    return _he(x)


# speedup vs baseline: 1.0591x; 1.0591x over previous
import jax
import jax.numpy as jnp
from jax import lax
from jax.experimental import pallas as pl
from jax.experimental.pallas import tpu as pltpu
from jax.experimental.pallas import tpu_sc as plsc

L = 16
NC = 2
NS = 16
B, C, H, W = 16, 3, 512, 512
NIMG = B * C
IMG_PER_CORE = NIMG // NC
G = 4
NGRP = IMG_PER_CORE // G
ROWS = H // NS
NK = W // L
NB = 256 // L


def _he_body(x_hbm, out_hbm, buf, hist, mhistg, mhist, merge, lut, lutbuf,
             shared_hist, shared_lut, in_sem, out_sem):
    cid = lax.axis_index("c")
    sid = lax.axis_index("s")
    row0 = sid * ROWS
    lane = lax.broadcasted_iota(jnp.int32, (L,), 0)
    lane_base = lane * 256
    ones = jnp.ones((L,), jnp.float32)
    zeros = jnp.zeros((L,), jnp.float32)

    def group(grp, _):
        img0 = cid * IMG_PER_CORE + grp * G

        for g in range(G):
            img = img0 + g
            bi, ci = img // C, img % C
            pv = img - G
            pb, pc = pv // C, pv % C

            @pl.when(grp > 0)
            def _():
                pltpu.make_async_copy(
                    buf.at[g], out_hbm.at[pb, pc, pl.ds(row0, ROWS)],
                    out_sem.at[g]).wait()

            pltpu.async_copy(
                x_hbm.at[bi, ci, pl.ds(row0, ROWS)], buf.at[g],
                in_sem.at[g])

        for g in range(G):
            img = img0 + g
            bi, ci = img // C, img % C
            pltpu.make_async_copy(
                x_hbm.at[bi, ci, pl.ds(row0, ROWS)], buf.at[g],
                in_sem.at[g]).wait()

            @plsc.parallel_loop(0, 512, unroll=8)
            def zero_hist(i):
                hist[pl.ds(i * L, L)] = zeros

            @plsc.parallel_loop(0, ROWS)
            def hist_row(r):
                @plsc.parallel_loop(0, NK // 2, unroll=4)
                def hist_px(k2):
                    for h in range(2):
                        v = buf[g, r, pl.ds((k2 * 2 + h) * L, L)]
                        xi = (v * 255.0).astype(jnp.int32)
                        plsc.addupdate_scatter(
                            hist, [lane_base + (xi + h * 4096)], ones)

            @plsc.parallel_loop(0, NB, unroll=2)
            def merge_lanes(c):
                acc = hist[pl.ds(c * L, L)]
                for r in range(1, 2 * L):
                    acc = acc + hist[pl.ds(r * 256 + c * L, L)]
                mhistg[g, pl.ds(c * L, L)] = acc

        pltpu.sync_copy(mhistg, shared_hist.at[sid])
        plsc.subcore_barrier()

        @pl.when(sid < G)
        def _():
            g = sid
            pltpu.sync_copy(shared_hist, merge)

            def merge_subcores(c, carry):
                tot, m = carry
                acc = merge[0, g, pl.ds(c * L, L)]
                for r in range(1, NS):
                    acc = acc + merge[r, g, pl.ds(c * L, L)]
                mhist[pl.ds(c * L, L)] = acc
                idx = lane + c * L
                comb = jnp.where(acc != 0.0,
                                 idx * 524288 + acc.astype(jnp.int32),
                                 -1)
                return tot + jnp.sum(acc), jnp.maximum(m, jnp.max(comb))

            tot, m = lax.fori_loop(
                0, NB, merge_subcores,
                (jnp.float32(0.0), jnp.int32(-1)))

            tot_v = jnp.full((L,), tot, jnp.float32)
            last_v = jnp.bitwise_and(jnp.full((L,), m, jnp.int32),
                                     524287).astype(jnp.float32)
            step_v = ((tot_v - last_v) / 255.0).astype(
                jnp.int32).astype(jnp.float32)
            half_v = (step_v * 0.5).astype(jnp.int32).astype(jnp.float32)
            safe_v = jnp.maximum(step_v, 1.0)
            is_id = step_v == 0.0

            def lut_chunk(c, carry_f):
                v = mhist[pl.ds(c * L, L)]
                excl = plsc.cumsum(v) + carry_f - v
                q = ((excl + half_v) / safe_v).astype(
                    jnp.int32).astype(jnp.float32)
                qc = jnp.clip(q, 0.0, 255.0)
                idx_f = (lane + c * L).astype(jnp.float32)
                lut[pl.ds(c * L, L)] = jnp.where(is_id, idx_f, qc) / 255.0
                return carry_f + jnp.sum(v)

            lax.fori_loop(0, NB, lut_chunk, jnp.float32(0.0))
            pltpu.sync_copy(lut, shared_lut.at[pl.ds(g * 256, 256)])

        plsc.subcore_barrier()
        pltpu.sync_copy(shared_lut, lutbuf)

        for g in range(G):
            img = img0 + g
            bi, ci = img // C, img % C

            @plsc.parallel_loop(0, ROWS)
            def gather_row(r):
                @plsc.parallel_loop(0, NK, unroll=8)
                def gather_px(k):
                    v = buf[g, r, pl.ds(k * L, L)]
                    xi = (v * 255.0).astype(jnp.int32) + g * 256
                    buf[g, r, pl.ds(k * L, L)] = plsc.load_gather(
                        lutbuf, [xi])

            pltpu.async_copy(
                buf.at[g], out_hbm.at[bi, ci, pl.ds(row0, ROWS)],
                out_sem.at[g])
        return 0

    lax.fori_loop(0, NGRP, group, 0)

    for g in range(G):
        img = cid * IMG_PER_CORE + (NGRP - 1) * G + g
        bi, ci = img // C, img % C
        pltpu.make_async_copy(
            buf.at[g], out_hbm.at[bi, ci, pl.ds(row0, ROWS)],
            out_sem.at[g]).wait()


_he = pl.kernel(
    _he_body,
    out_type=jax.ShapeDtypeStruct((B, C, H, W), jnp.float32),
    mesh=plsc.VectorSubcoreMesh(core_axis_name="c", subcore_axis_name="s"),
    compiler_params=pltpu.CompilerParams(
        needs_layout_passes=False, use_tc_tiling_on_sc=True),
    scratch_types=[
        pltpu.VMEM((G, ROWS, W), jnp.float32),
        pltpu.VMEM((2 * L * 256,), jnp.float32),
        pltpu.VMEM((G, 256), jnp.float32),
        pltpu.VMEM((256,), jnp.float32),
        pltpu.VMEM((NS, G, 256), jnp.float32),
        pltpu.VMEM((256,), jnp.float32),
        pltpu.VMEM((G * 256,), jnp.float32),
        pltpu.VMEM_SHARED((NS, G, 256), jnp.float32),
        pltpu.VMEM_SHARED((G * 256,), jnp.float32),
        pltpu.SemaphoreType.DMA((G,)),
        pltpu.SemaphoreType.DMA((G,)),
    ],
)


def kernel(x):
    return _he(x)


# speedup vs baseline: 1.2714x; 1.2005x over previous
import jax
import jax.numpy as jnp
from jax import lax
from jax.experimental import pallas as pl
from jax.experimental.pallas import tpu as pltpu
from jax.experimental.pallas import tpu_sc as plsc

L = 16
NC = 2
NS = 16
B, C, H, W = 16, 3, 512, 512
NIMG = B * C
IMG_PER_CORE = NIMG // NC
G = 4
NGRP = IMG_PER_CORE // G
ROWS = H // NS
NK = W // L
NB = 256 // L


def _he_body(x_hbm, out_hbm, buf, hist, mhistg, mhist, merge, lut, lutbuf,
             shared_hist, shared_lut, in_sem, out_sem):
    cid = lax.axis_index("c")
    sid = lax.axis_index("s")
    row0 = sid * ROWS
    lane = lax.broadcasted_iota(jnp.int32, (L,), 0)
    lane_base = lane * 256
    ones = jnp.ones((L,), jnp.float32)
    zeros = jnp.zeros((L,), jnp.float32)

    def group(grp, _):
        img0 = cid * IMG_PER_CORE + grp * G

        for g in range(G):
            img = img0 + g
            bi, ci = img // C, img % C
            pv = img - G
            pb, pc = pv // C, pv % C

            @pl.when(grp > 0)
            def _():
                pltpu.make_async_copy(
                    buf.at[g], out_hbm.at[pb, pc, pl.ds(row0, ROWS)],
                    out_sem.at[g]).wait()

            pltpu.async_copy(
                x_hbm.at[bi, ci, pl.ds(row0, ROWS)], buf.at[g],
                in_sem.at[g])

        for g in range(G):
            img = img0 + g
            bi, ci = img // C, img % C
            pltpu.make_async_copy(
                x_hbm.at[bi, ci, pl.ds(row0, ROWS)], buf.at[g],
                in_sem.at[g]).wait()

            @plsc.parallel_loop(0, 256, unroll=8)
            def zero_hist(i):
                hist[pl.ds(i * L, L)] = zeros

            @plsc.parallel_loop(0, ROWS * NK, unroll=8)
            def hist_px(i):
                r = i >> 5
                k = i & (NK - 1)
                v = buf[g, r, pl.ds(k * L, L)]
                xi = (v * 255.0).astype(jnp.int32)
                plsc.addupdate_scatter(hist, [lane_base + xi], ones)

            @plsc.parallel_loop(0, NB, unroll=2)
            def merge_lanes(c):
                acc = hist[pl.ds(c * L, L)]
                for r in range(1, L):
                    acc = acc + hist[pl.ds(r * 256 + c * L, L)]
                mhistg[g, pl.ds(c * L, L)] = acc

        pltpu.sync_copy(mhistg, shared_hist.at[sid])
        plsc.subcore_barrier()

        @pl.when(sid < G)
        def _():
            g = sid
            pltpu.sync_copy(shared_hist, merge)

            def merge_subcores(c, carry):
                tot, m = carry
                acc = merge[0, g, pl.ds(c * L, L)]
                for r in range(1, NS):
                    acc = acc + merge[r, g, pl.ds(c * L, L)]
                mhist[pl.ds(c * L, L)] = acc
                idx = lane + c * L
                comb = jnp.where(acc != 0.0,
                                 idx * 524288 + acc.astype(jnp.int32),
                                 -1)
                return tot + jnp.sum(acc), jnp.maximum(m, jnp.max(comb))

            tot, m = lax.fori_loop(
                0, NB, merge_subcores,
                (jnp.float32(0.0), jnp.int32(-1)))

            tot_v = jnp.full((L,), tot, jnp.float32)
            last_v = jnp.bitwise_and(jnp.full((L,), m, jnp.int32),
                                     524287).astype(jnp.float32)
            step_v = ((tot_v - last_v) / 255.0).astype(
                jnp.int32).astype(jnp.float32)
            half_v = (step_v * 0.5).astype(jnp.int32).astype(jnp.float32)
            safe_v = jnp.maximum(step_v, 1.0)
            is_id = step_v == 0.0

            def lut_chunk(c, carry_f):
                v = mhist[pl.ds(c * L, L)]
                excl = plsc.cumsum(v) + carry_f - v
                q = ((excl + half_v) / safe_v).astype(
                    jnp.int32).astype(jnp.float32)
                qc = jnp.clip(q, 0.0, 255.0)
                idx_f = (lane + c * L).astype(jnp.float32)
                lut[pl.ds(c * L, L)] = jnp.where(is_id, idx_f, qc) / 255.0
                return carry_f + jnp.sum(v)

            lax.fori_loop(0, NB, lut_chunk, jnp.float32(0.0))
            pltpu.sync_copy(lut, shared_lut.at[pl.ds(g * 256, 256)])

        plsc.subcore_barrier()
        pltpu.sync_copy(shared_lut, lutbuf)

        for g in range(G):
            img = img0 + g
            bi, ci = img // C, img % C

            @plsc.parallel_loop(0, ROWS * NK, unroll=8)
            def gather_px(i):
                r = i >> 5
                k = i & (NK - 1)
                v = buf[g, r, pl.ds(k * L, L)]
                xi = (v * 255.0).astype(jnp.int32) + g * 256
                buf[g, r, pl.ds(k * L, L)] = plsc.load_gather(lutbuf, [xi])

            pltpu.async_copy(
                buf.at[g], out_hbm.at[bi, ci, pl.ds(row0, ROWS)],
                out_sem.at[g])
        return 0

    lax.fori_loop(0, NGRP, group, 0)

    for g in range(G):
        img = cid * IMG_PER_CORE + (NGRP - 1) * G + g
        bi, ci = img // C, img % C
        pltpu.make_async_copy(
            buf.at[g], out_hbm.at[bi, ci, pl.ds(row0, ROWS)],
            out_sem.at[g]).wait()


_he = pl.kernel(
    _he_body,
    out_type=jax.ShapeDtypeStruct((B, C, H, W), jnp.float32),
    mesh=plsc.VectorSubcoreMesh(core_axis_name="c", subcore_axis_name="s"),
    compiler_params=pltpu.CompilerParams(
        needs_layout_passes=False, use_tc_tiling_on_sc=True),
    scratch_types=[
        pltpu.VMEM((G, ROWS, W), jnp.float32),
        pltpu.VMEM((L * 256,), jnp.float32),
        pltpu.VMEM((G, 256), jnp.float32),
        pltpu.VMEM((256,), jnp.float32),
        pltpu.VMEM((NS, G, 256), jnp.float32),
        pltpu.VMEM((256,), jnp.float32),
        pltpu.VMEM((G * 256,), jnp.float32),
        pltpu.VMEM_SHARED((NS, G, 256), jnp.float32),
        pltpu.VMEM_SHARED((G * 256,), jnp.float32),
        pltpu.SemaphoreType.DMA((G,)),
        pltpu.SemaphoreType.DMA((G,)),
    ],
)


def kernel(x):
    return _he(x)


# speedup vs baseline: 1.4501x; 1.1405x over previous
import jax
import jax.numpy as jnp
from jax import lax
from jax.experimental import pallas as pl
from jax.experimental.pallas import tpu as pltpu
from jax.experimental.pallas import tpu_sc as plsc

L = 16
NC = 2
NS = 16
B, C, H, W = 16, 3, 512, 512
NIMG = B * C
IMG_PER_CORE = NIMG // NC
G = 4
NGRP = IMG_PER_CORE // G
ROWS = H // NS
NK = W // L
NB = 256 // L


def _he_body(x_hbm, out_hbm, buf, obuf, hist, mhistg, mhist, merge, lut,
             lutbuf, shared_hist, shared_lut, in_sem, out_sem):
    cid = lax.axis_index("c")
    sid = lax.axis_index("s")
    row0 = sid * ROWS
    lane = lax.broadcasted_iota(jnp.int32, (L,), 0)
    lane_base = lane * 256
    ones = jnp.ones((L,), jnp.float32)
    zeros = jnp.zeros((L,), jnp.float32)

    for g in range(G):
        img = cid * IMG_PER_CORE + g
        bi, ci = img // C, img % C
        pltpu.async_copy(
            x_hbm.at[bi, ci, pl.ds(row0, ROWS)], buf.at[g], in_sem.at[g])

    def group(grp, _):
        img0 = cid * IMG_PER_CORE + grp * G

        for g in range(G):
            img = img0 + g
            bi, ci = img // C, img % C
            pltpu.make_async_copy(
                x_hbm.at[bi, ci, pl.ds(row0, ROWS)], buf.at[g],
                in_sem.at[g]).wait()

            @plsc.parallel_loop(0, 256, unroll=8)
            def zero_hist(i):
                hist[pl.ds(i * L, L)] = zeros

            @plsc.parallel_loop(0, ROWS * NK, unroll=8)
            def hist_px(i):
                r = i >> 5
                k = i & (NK - 1)
                v = buf[g, r, pl.ds(k * L, L)]
                xi = (v * 255.0).astype(jnp.int32)
                plsc.addupdate_scatter(hist, [lane_base + xi], ones)

            @plsc.parallel_loop(0, NB, unroll=2)
            def merge_lanes(c):
                acc = hist[pl.ds(c * L, L)]
                for r in range(1, L):
                    acc = acc + hist[pl.ds(r * 256 + c * L, L)]
                mhistg[g, pl.ds(c * L, L)] = acc

        pltpu.sync_copy(mhistg, shared_hist.at[sid])
        plsc.subcore_barrier()

        @pl.when(sid < G)
        def _():
            g = sid
            pltpu.sync_copy(shared_hist, merge)

            def merge_subcores(c, carry):
                tot, m = carry
                acc = merge[0, g, pl.ds(c * L, L)]
                for r in range(1, NS):
                    acc = acc + merge[r, g, pl.ds(c * L, L)]
                mhist[pl.ds(c * L, L)] = acc
                idx = lane + c * L
                comb = jnp.where(acc != 0.0,
                                 idx * 524288 + acc.astype(jnp.int32),
                                 -1)
                return tot + jnp.sum(acc), jnp.maximum(m, jnp.max(comb))

            tot, m = lax.fori_loop(
                0, NB, merge_subcores,
                (jnp.float32(0.0), jnp.int32(-1)))

            tot_v = jnp.full((L,), tot, jnp.float32)
            last_v = jnp.bitwise_and(jnp.full((L,), m, jnp.int32),
                                     524287).astype(jnp.float32)
            step_v = ((tot_v - last_v) / 255.0).astype(
                jnp.int32).astype(jnp.float32)
            half_v = (step_v * 0.5).astype(jnp.int32).astype(jnp.float32)
            safe_v = jnp.maximum(step_v, 1.0)
            is_id = step_v == 0.0

            def lut_chunk(c, carry_f):
                v = mhist[pl.ds(c * L, L)]
                excl = plsc.cumsum(v) + carry_f - v
                q = ((excl + half_v) / safe_v).astype(
                    jnp.int32).astype(jnp.float32)
                qc = jnp.clip(q, 0.0, 255.0)
                idx_f = (lane + c * L).astype(jnp.float32)
                lut[pl.ds(c * L, L)] = jnp.where(is_id, idx_f, qc) / 255.0
                return carry_f + jnp.sum(v)

            lax.fori_loop(0, NB, lut_chunk, jnp.float32(0.0))
            pltpu.sync_copy(lut, shared_lut.at[pl.ds(g * 256, 256)])

        plsc.subcore_barrier()
        pltpu.sync_copy(shared_lut, lutbuf)

        for g in range(G):
            img = img0 + g
            bi, ci = img // C, img % C
            rb = g & 1

            if g >= 2:
                pltpu.make_async_copy(
                    obuf.at[rb], out_hbm.at[bi, ci, pl.ds(row0, ROWS)],
                    out_sem.at[rb]).wait()
            else:
                @pl.when(grp > 0)
                def _():
                    pltpu.make_async_copy(
                        obuf.at[rb], out_hbm.at[bi, ci, pl.ds(row0, ROWS)],
                        out_sem.at[rb]).wait()

            @plsc.parallel_loop(0, ROWS * NK, unroll=8)
            def gather_px(i):
                r = i >> 5
                k = i & (NK - 1)
                v = buf[g, r, pl.ds(k * L, L)]
                xi = (v * 255.0).astype(jnp.int32) + g * 256
                obuf[rb, r, pl.ds(k * L, L)] = plsc.load_gather(
                    lutbuf, [xi])

            pltpu.async_copy(
                obuf.at[rb], out_hbm.at[bi, ci, pl.ds(row0, ROWS)],
                out_sem.at[rb])

            nimg = img + G
            nb, ncl = nimg // C, nimg % C

            @pl.when(grp < NGRP - 1)
            def _():
                pltpu.async_copy(
                    x_hbm.at[nb, ncl, pl.ds(row0, ROWS)], buf.at[g],
                    in_sem.at[g])
        return 0

    lax.fori_loop(0, NGRP, group, 0)

    for rb in range(2):
        img = cid * IMG_PER_CORE + (NGRP - 1) * G + 2 + rb
        bi, ci = img // C, img % C
        pltpu.make_async_copy(
            obuf.at[rb], out_hbm.at[bi, ci, pl.ds(row0, ROWS)],
            out_sem.at[rb]).wait()


_he = pl.kernel(
    _he_body,
    out_type=jax.ShapeDtypeStruct((B, C, H, W), jnp.float32),
    mesh=plsc.VectorSubcoreMesh(core_axis_name="c", subcore_axis_name="s"),
    compiler_params=pltpu.CompilerParams(
        needs_layout_passes=False, use_tc_tiling_on_sc=True),
    scratch_types=[
        pltpu.VMEM((G, ROWS, W), jnp.float32),
        pltpu.VMEM((2, ROWS, W), jnp.float32),
        pltpu.VMEM((L * 256,), jnp.float32),
        pltpu.VMEM((G, 256), jnp.float32),
        pltpu.VMEM((256,), jnp.float32),
        pltpu.VMEM((NS, G, 256), jnp.float32),
        pltpu.VMEM((256,), jnp.float32),
        pltpu.VMEM((G * 256,), jnp.float32),
        pltpu.VMEM_SHARED((NS, G, 256), jnp.float32),
        pltpu.VMEM_SHARED((G * 256,), jnp.float32),
        pltpu.SemaphoreType.DMA((G,)),
        pltpu.SemaphoreType.DMA((2,)),
    ],
)


def kernel(x):
    return _he(x)
